# baseline (speedup 1.0000x reference)
import functools

import jax
import jax.numpy as jnp
from jax import lax
from jax.experimental import pallas as pl
from jax.experimental.pallas import tpu as pltpu
from jax.experimental.pallas import tpu_sc as plsc

_NC = 2
_NS = 16
_NW = _NC * _NS
_LANES = 16


def _make_sc_broadcast(S1, B, V, dtype):
    mesh = plsc.VectorSubcoreMesh(core_axis_name="c", subcore_axis_name="s")
    n_blocks = S1
    per_core = (n_blocks + _NC - 1) // _NC
    max_per_tile = (per_core + _NS - 1) // _NS
    half = B // 2
    n_full = V // _LANES
    tail = V - n_full * _LANES

    @functools.partial(
        pl.kernel,
        mesh=mesh,
        out_type=jax.ShapeDtypeStruct((S1, B, V), dtype),
        scratch_types=[
            pltpu.VMEM((V,), dtype),
            pltpu.VMEM((half, V), dtype),
            pltpu.SemaphoreType.DMA,
        ],
    )
    def sc_broadcast(logs_hbm, out_hbm, logs_v, buf, sem):
        cid = lax.axis_index("c")
        sid = lax.axis_index("s")
        pltpu.sync_copy(logs_hbm, logs_v)

        chunks = [
            logs_v[pl.ds(i * _LANES, _LANES)] for i in range(n_full)
        ]
        if tail:
            chunks.append(logs_v[pl.ds(V - _LANES, _LANES)])

        def fill_row(r, carry):
            for i in range(n_full):
                buf[r, pl.ds(i * _LANES, _LANES)] = chunks[i]
            if tail:
                buf[r, pl.ds(V - _LANES, _LANES)] = chunks[-1]
            return carry

        lax.fori_loop(0, half, fill_row, 0)

        base = cid * per_core
        for j in range(max_per_tile):
            local = sid + j * _NS
            step = base + local
            @pl.when(step < n_blocks)
            def _():
                pltpu.make_async_copy(
                    buf, out_hbm.at[step, pl.ds(0, half)], sem).start()
                pltpu.make_async_copy(
                    buf, out_hbm.at[step, pl.ds(half, half)], sem).start()
        for j in range(max_per_tile):
            local = sid + j * _NS
            step = base + local
            @pl.when(step < n_blocks)
            def _():
                pltpu.make_async_copy(
                    buf, out_hbm.at[step, pl.ds(0, half)], sem).wait()
                pltpu.make_async_copy(
                    buf, out_hbm.at[step, pl.ds(half, half)], sem).wait()

    return sc_broadcast


def kernel(hist, logs):
    S_, B_ = hist.shape
    V = logs.shape[0]
    fn = _make_sc_broadcast(S_ + 1, B_, V, logs.dtype)
    return fn(logs)


# speedup vs baseline: 1.0077x; 1.0077x over previous
import functools

import jax
import jax.numpy as jnp
from jax import lax
from jax.experimental import pallas as pl
from jax.experimental.pallas import tpu as pltpu
from jax.experimental.pallas import tpu_sc as plsc

_NC = 2
_NS = 16
_NW = _NC * _NS
_LANES = 16


def _make_sc_broadcast(S1, B, V, dtype):
    mesh = plsc.VectorSubcoreMesh(core_axis_name="c", subcore_axis_name="s")
    n_blocks = S1
    per_core = (n_blocks + _NC - 1) // _NC
    max_per_tile = (per_core + _NS - 1) // _NS
    half = B // 2
    n_full = V // _LANES
    tail = V - n_full * _LANES

    @functools.partial(
        pl.kernel,
        mesh=mesh,
        out_type=jax.ShapeDtypeStruct((S1, B, V), dtype),
        scratch_types=[
            pltpu.VMEM((V,), dtype),
            pltpu.VMEM((half, V), dtype),
            pltpu.SemaphoreType.DMA,
        ],
    )
    def sc_broadcast(logs_hbm, out_hbm, logs_v, buf, sem):
        cid = lax.axis_index("c")
        sid = lax.axis_index("s")
        pltpu.sync_copy(logs_hbm, logs_v)

        chunks = [
            logs_v[pl.ds(i * _LANES, _LANES)] for i in range(n_full)
        ]
        if tail:
            chunks.append(logs_v[pl.ds(V - _LANES, _LANES)])

        def fill_row(r, carry):
            for i in range(n_full):
                buf[r, pl.ds(i * _LANES, _LANES)] = chunks[i]
            if tail:
                buf[r, pl.ds(V - _LANES, _LANES)] = chunks[-1]
            return carry

        lax.fori_loop(0, half, fill_row, 0)

        base = cid * per_core
        for j in range(max_per_tile):
            local = sid + j * _NS
            step = base + local
            @pl.when(jnp.logical_and(local < per_core, step < n_blocks))
            def _():
                pltpu.make_async_copy(
                    buf, out_hbm.at[step, pl.ds(0, half)], sem).start()
                pltpu.make_async_copy(
                    buf, out_hbm.at[step, pl.ds(half, half)], sem).start()
        for j in range(max_per_tile):
            local = sid + j * _NS
            step = base + local
            @pl.when(jnp.logical_and(local < per_core, step < n_blocks))
            def _():
                pltpu.make_async_copy(
                    buf, out_hbm.at[step, pl.ds(0, half)], sem).wait()
                pltpu.make_async_copy(
                    buf, out_hbm.at[step, pl.ds(half, half)], sem).wait()

    return sc_broadcast


def kernel(hist, logs):
    S_, B_ = hist.shape
    V = logs.shape[0]
    fn = _make_sc_broadcast(S_ + 1, B_, V, logs.dtype)
    return fn(logs)
